# baseline (speedup 1.0000x reference)
import jax
import jax.numpy as jnp
from jax.experimental import pallas as pl
from jax.experimental.pallas import tpu as pltpu

B, F_IN, F_OUT = 64, 4096, 4096
OB = 512


def _mm_body(x_ref, w_ref, m_ref, o_ref):
    wm = w_ref[...] * m_ref[...].astype(jnp.float32)
    o_ref[...] = jax.lax.dot_general(
        x_ref[...], wm, (((1,), (1,)), ((), ())),
        preferred_element_type=jnp.float32)


def kernel(x, weight, mask):
    m4 = mask.astype(jnp.int2)
    grid = (F_OUT // OB,)
    return pl.pallas_call(
        _mm_body,
        grid=grid,
        in_specs=[
            pl.BlockSpec((B, F_IN), lambda o: (0, 0)),
            pl.BlockSpec((OB, F_IN), lambda o: (o, 0)),
            pl.BlockSpec((OB, F_IN), lambda o: (o, 0)),
        ],
        out_specs=pl.BlockSpec((B, OB), lambda o: (0, o)),
        out_shape=jax.ShapeDtypeStruct((B, F_OUT), jnp.float32),
        compiler_params=pltpu.CompilerParams(
            dimension_semantics=("arbitrary",)),
    )(x, weight, m4)
